# baseline (speedup 1.0000x reference)
import functools
import math

import jax
import jax.numpy as jnp
from jax import lax
from jax.experimental import pallas as pl
from jax.experimental.pallas import tpu as pltpu
from jax.experimental.pallas import tpu_sc as plsc

N = 8388608
T = 16
B = 1801
NB = T * B
NC = 2
NS = 16
NW = NC * NS
CHUNK = N // NW
TILE_E = 8192
STEPS = CHUNK // TILE_E
VECS = TILE_E // 16
IROWS = TILE_E // 128

_SCALE = 1800.0 / math.pi
_C = [c * _SCALE for c in (
    1.5707963050, -0.2145988016, 0.0889789874, -0.0501743046,
    0.0308918810, -0.0170881256, 0.0066700901, -0.0012624911)]

_mesh = plsc.VectorSubcoreMesh(core_axis_name="c", subcore_axis_name="s")


@functools.partial(
    pl.kernel,
    mesh=_mesh,
    out_type=jax.ShapeDtypeStruct((NC, NB), jnp.float32),
    scratch_types=[
        pltpu.VMEM((TILE_E,), jnp.float32),
        pltpu.VMEM((TILE_E,), jnp.int32),
        pltpu.VMEM((TILE_E,), jnp.float32),
        pltpu.VMEM((TILE_E,), jnp.int32),
        pltpu.VMEM((IROWS, 128), jnp.int32),
        pltpu.VMEM((IROWS, 128), jnp.int32),
        pltpu.VMEM((128,), jnp.float32),
        pltpu.VMEM((NB,), jnp.float32),
        pltpu.VMEM_SHARED((NB,), jnp.float32),
        pltpu.SemaphoreType.DMA,
        pltpu.SemaphoreType.DMA,
        pltpu.SemaphoreType.DMA,
        pltpu.SemaphoreType.DMA,
    ],
    compiler_params=pltpu.CompilerParams(needs_layout_passes=False),
)
def _hist_sc(prods_hbm, atoms_hbm, out_hbm,
             pbuf0, abuf0, pbuf1, abuf1, ibuf0, ibuf1, ones_v, zbuf,
             shared_hist, sem0, sem1, ssem0, ssem1):
    cid = lax.axis_index("c")
    sid = lax.axis_index("s")
    base = (cid * NS + sid) * CHUNK
    zeros16 = jnp.zeros((16,), jnp.float32)
    ones16 = jnp.ones((16,), jnp.float32)

    def start_in(st, pb, ab, sem):
        off = base + st * TILE_E
        pltpu.make_async_copy(prods_hbm.at[pl.ds(off, TILE_E)], pb, sem).start()
        pltpu.make_async_copy(atoms_hbm.at[pl.ds(off, TILE_E)], ab, sem).start()

    def drain_in(st, pb, ab, sem):
        off = base + st * TILE_E
        pltpu.make_async_copy(prods_hbm.at[pl.ds(off, TILE_E)], pb, sem).wait()
        pltpu.make_async_copy(atoms_hbm.at[pl.ds(off, TILE_E)], ab, sem).wait()

    def issue_streams(ib, sem):
        def go(k, carry):
            pltpu.make_async_copy(ones_v, shared_hist.at[ib.at[k]], sem
                                  ).start(add=True)
            return carry
        lax.fori_loop(0, IROWS, go, 0)

    def drain_streams(ib, sem):
        def go(k, carry):
            pltpu.make_async_copy(ones_v, shared_hist.at[ib.at[k]], sem).wait()
            return carry
        lax.fori_loop(0, IROWS, go, 0)

    start_in(0, pbuf0, abuf0, sem0)

    for k in range(8):
        ones_v[pl.ds(k * 16, 16)] = ones16

    @pl.when(sid == 0)
    def _():
        def zero_body(i, carry):
            zbuf[pl.ds(i * 16, 16)] = zeros16
            return carry
        lax.fori_loop(0, NB // 16, zero_body, 0)
        pltpu.sync_copy(zbuf, shared_hist)

    plsc.subcore_barrier()

    def compute(pbuf, abuf, ibuf, j):
        p = pbuf[pl.ds(j * 16, 16)]
        a = abuf[pl.ds(j * 16, 16)]
        p = jnp.minimum(jnp.maximum(p, 0.0), 1.0 - 1e-6)
        x = 1.0 - p
        xh = 0.5 * x
        yi = jnp.int32(0x5F3759DF) - (lax.bitcast_convert_type(x, jnp.int32) >> 1)
        y = lax.bitcast_convert_type(yi, jnp.float32)
        y = y * (1.5 - xh * y * y)
        y = y * (1.5 - xh * y * y)
        y = y * (1.5 - xh * y * y)
        s = x * y
        poly = jnp.float32(_C[7])
        for c in (_C[6], _C[5], _C[4], _C[3], _C[2], _C[1], _C[0]):
            poly = poly * p + jnp.float32(c)
        binf = s * poly + 0.5
        bin_i = binf.astype(jnp.int32)
        bin_i = jnp.minimum(jnp.maximum(bin_i, 0), B - 1)
        idx = a * B + bin_i
        ibuf[j >> 3, pl.ds((j & 7) * 16, 16)] = idx

    def outer(i, carry):
        st0 = 2 * i
        drain_in(st0, pbuf0, abuf0, sem0)
        start_in(st0 + 1, pbuf1, abuf1, sem1)

        @pl.when(i > 0)
        def _():
            drain_streams(ibuf0, ssem0)

        plsc.parallel_loop(0, VECS, 1, unroll=16)(
            functools.partial(compute, pbuf0, abuf0, ibuf0))
        issue_streams(ibuf0, ssem0)

        drain_in(st0 + 1, pbuf1, abuf1, sem1)

        @pl.when(i < STEPS // 2 - 1)
        def _():
            start_in(st0 + 2, pbuf0, abuf0, sem0)

        @pl.when(i > 0)
        def _():
            drain_streams(ibuf1, ssem1)

        plsc.parallel_loop(0, VECS, 1, unroll=16)(
            functools.partial(compute, pbuf1, abuf1, ibuf1))
        issue_streams(ibuf1, ssem1)
        return carry

    lax.fori_loop(0, STEPS // 2, outer, 0)

    drain_streams(ibuf0, ssem0)
    drain_streams(ibuf1, ssem1)
    plsc.subcore_barrier()

    @pl.when(sid == 0)
    def _():
        pltpu.sync_copy(shared_hist, out_hbm.at[cid])


def _finish_tc(partials_ref, tgt_ref, probs_ref, vw_ref, out_ref):
    hist = jnp.sum(partials_ref[...], axis=0)
    s = jnp.sum(hist, axis=1, keepdims=True)
    s = jnp.where(s == 0.0, 1.0, s)
    d = hist / s - tgt_ref[...]
    k = 1
    while k < B:
        shifted = jnp.concatenate(
            [jnp.zeros((T, k), jnp.float32), d[:, : B - k]], axis=1)
        d = d + shifted
        k *= 2
    w1 = jnp.sum(jnp.abs(d), axis=1, keepdims=True) * 0.1
    pw = probs_ref[...] * vw_ref[...]
    total = jnp.sum(w1 * pw) / (jnp.sum(pw) + 1e-5)
    out_ref[...] = jnp.reshape(total, (1, 1))


def kernel(prods, atom_types, target_angles, atom_types_probabilities, valency_weight):
    partials = _hist_sc(prods, atom_types)
    res = pl.pallas_call(
        _finish_tc,
        out_shape=jax.ShapeDtypeStruct((1, 1), jnp.float32),
    )(
        partials.reshape(NC, T, B),
        target_angles,
        atom_types_probabilities.reshape(T, 1),
        valency_weight.reshape(T, 1),
    )
    return res[0, 0]


# speedup vs baseline: 1.1822x; 1.1822x over previous
import functools
import math

import jax
import jax.numpy as jnp
from jax import lax
from jax.experimental import pallas as pl
from jax.experimental.pallas import tpu as pltpu
from jax.experimental.pallas import tpu_sc as plsc

N = 8388608
T = 16
B = 1801
NB = T * B
NC = 2
NS = 16
NW = NC * NS
CHUNK = N // NW
TILE_E = 8192
STEPS = CHUNK // TILE_E
VECS = TILE_E // 16
IROWS = TILE_E // 128

_SCALE = 1800.0 / math.pi
_C = [c * _SCALE for c in (
    1.5707963050, -0.2145988016, 0.0889789874, -0.0501743046,
    0.0308918810, -0.0170881256, 0.0066700901, -0.0012624911)]

_mesh = plsc.VectorSubcoreMesh(core_axis_name="c", subcore_axis_name="s")


@functools.partial(
    pl.kernel,
    mesh=_mesh,
    out_type=[jax.ShapeDtypeStruct((NB,), jnp.float32),
              jax.ShapeDtypeStruct((NB,), jnp.float32)],
    scratch_types=[
        pltpu.VMEM((TILE_E,), jnp.float32),
        pltpu.VMEM((TILE_E,), jnp.int32),
        pltpu.VMEM((TILE_E,), jnp.float32),
        pltpu.VMEM((TILE_E,), jnp.int32),
        pltpu.VMEM((IROWS, 128), jnp.int32),
        pltpu.VMEM((IROWS, 128), jnp.int32),
        pltpu.VMEM((128,), jnp.float32),
        pltpu.VMEM((NB,), jnp.float32),
        pltpu.VMEM_SHARED((NB,), jnp.float32),
        pltpu.SemaphoreType.DMA,
        pltpu.SemaphoreType.DMA,
        pltpu.SemaphoreType.DMA,
        pltpu.SemaphoreType.DMA,
    ],
    compiler_params=pltpu.CompilerParams(needs_layout_passes=False),
)
def _hist_sc(prods_hbm, atoms_hbm, out0_hbm, out1_hbm,
             pbuf0, abuf0, pbuf1, abuf1, ibuf0, ibuf1, ones_v, zbuf,
             shared_hist, sem0, sem1, ssem0, ssem1):
    cid = lax.axis_index("c")
    sid = lax.axis_index("s")
    base = (cid * NS + sid) * CHUNK
    zeros16 = jnp.zeros((16,), jnp.float32)
    ones16 = jnp.ones((16,), jnp.float32)

    def start_in(st, pb, ab, sem):
        off = base + st * TILE_E
        pltpu.make_async_copy(prods_hbm.at[pl.ds(off, TILE_E)], pb, sem).start()
        pltpu.make_async_copy(atoms_hbm.at[pl.ds(off, TILE_E)], ab, sem).start()

    def drain_in(st, pb, ab, sem):
        off = base + st * TILE_E
        pltpu.make_async_copy(prods_hbm.at[pl.ds(off, TILE_E)], pb, sem).wait()
        pltpu.make_async_copy(atoms_hbm.at[pl.ds(off, TILE_E)], ab, sem).wait()

    def issue_rows(ib, sem, k0, nk):
        def go(k, carry):
            pltpu.make_async_copy(ones_v, shared_hist.at[ib.at[k]], sem
                                  ).start(add=True)
            return carry
        lax.fori_loop(k0, k0 + nk, go, 0)

    def drain_streams(ib, sem):
        def go(k, carry):
            pltpu.make_async_copy(ones_v, shared_hist.at[ib.at[k]], sem
                                  ).wait()
            return carry
        lax.fori_loop(0, IROWS, go, 0)

    start_in(0, pbuf0, abuf0, sem0)

    for k in range(8):
        ones_v[pl.ds(k * 16, 16)] = ones16

    @pl.when(sid == 0)
    def _():
        def zero_body(i, carry):
            zbuf[pl.ds(i * 16, 16)] = zeros16
            return carry
        lax.fori_loop(0, NB // 16, zero_body, 0)
        pltpu.sync_copy(zbuf, shared_hist)

    plsc.subcore_barrier()

    def compute(pbuf, abuf, ibuf, j):
        p = pbuf[pl.ds(j * 16, 16)]
        a = abuf[pl.ds(j * 16, 16)]
        p = jnp.minimum(jnp.maximum(p, 0.0), 1.0 - 1e-6)
        x = 1.0 - p
        xh = 0.5 * x
        yi = jnp.int32(0x5F3759DF) - (lax.bitcast_convert_type(x, jnp.int32) >> 1)
        y = lax.bitcast_convert_type(yi, jnp.float32)
        y = y * (1.5 - xh * y * y)
        y = y * (1.5 - xh * y * y)
        y = y * (1.5 - xh * y * y)
        s = x * y
        poly = jnp.float32(_C[7])
        for c in (_C[6], _C[5], _C[4], _C[3], _C[2], _C[1], _C[0]):
            poly = poly * p + jnp.float32(c)
        binf = s * poly + 0.5
        bin_i = binf.astype(jnp.int32)
        bin_i = jnp.minimum(jnp.maximum(bin_i, 0), B - 1)
        idx = a * B + bin_i
        ibuf[j >> 3, pl.ds((j & 7) * 16, 16)] = idx

    def outer(i, carry):
        st0 = 2 * i
        drain_in(st0, pbuf0, abuf0, sem0)
        start_in(st0 + 1, pbuf1, abuf1, sem1)

        @pl.when(i > 0)
        def _():
            drain_streams(ibuf0, ssem0)

        def group0(g, carry):
            plsc.parallel_loop(g * 64, (g + 1) * 64, 1, unroll=16)(
                functools.partial(compute, pbuf0, abuf0, ibuf0))
            issue_rows(ibuf0, ssem0, g * 8, 8)
            return carry

        lax.fori_loop(0, VECS // 64, group0, 0)

        drain_in(st0 + 1, pbuf1, abuf1, sem1)

        @pl.when(i < STEPS // 2 - 1)
        def _():
            start_in(st0 + 2, pbuf0, abuf0, sem0)

        @pl.when(i > 0)
        def _():
            drain_streams(ibuf1, ssem1)

        def group1(g, carry):
            plsc.parallel_loop(g * 64, (g + 1) * 64, 1, unroll=16)(
                functools.partial(compute, pbuf1, abuf1, ibuf1))
            issue_rows(ibuf1, ssem1, g * 8, 8)
            return carry

        lax.fori_loop(0, VECS // 64, group1, 0)
        return carry

    lax.fori_loop(0, STEPS // 2, outer, 0)

    drain_streams(ibuf0, ssem0)
    drain_streams(ibuf1, ssem1)
    plsc.subcore_barrier()

    @pl.when(jnp.logical_and(sid == 0, cid == 0))
    def _():
        pltpu.sync_copy(shared_hist, out0_hbm)

    @pl.when(jnp.logical_and(sid == 0, cid == 1))
    def _():
        pltpu.sync_copy(shared_hist, out1_hbm)


def _finish_tc(partials_ref, tgt_ref, probs_ref, vw_ref, out_ref):
    hist = jnp.sum(partials_ref[...], axis=0)
    s = jnp.sum(hist, axis=1, keepdims=True)
    s = jnp.where(s == 0.0, 1.0, s)
    d = hist / s - tgt_ref[...]
    k = 1
    while k < B:
        shifted = jnp.concatenate(
            [jnp.zeros((T, k), jnp.float32), d[:, : B - k]], axis=1)
        d = d + shifted
        k *= 2
    w1 = jnp.sum(jnp.abs(d), axis=1, keepdims=True) * 0.1
    pw = probs_ref[...] * vw_ref[...]
    total = jnp.sum(w1 * pw) / (jnp.sum(pw) + 1e-5)
    out_ref[...] = jnp.reshape(total, (1, 1))


def kernel(prods, atom_types, target_angles, atom_types_probabilities, valency_weight):
    p0, p1 = _hist_sc(prods, atom_types)
    partials = jnp.stack([p0, p1])
    res = pl.pallas_call(
        _finish_tc,
        out_shape=jax.ShapeDtypeStruct((1, 1), jnp.float32),
    )(
        partials.reshape(NC, T, B),
        target_angles,
        atom_types_probabilities.reshape(T, 1),
        valency_weight.reshape(T, 1),
    )
    return res[0, 0]
